# speedup vs baseline: 2.4081x; 2.4081x over previous
import functools
import math

import jax
import jax.numpy as jnp
from jax import lax
from jax.experimental import pallas as pl
from jax.experimental.pallas import tpu as pltpu
from jax.experimental.pallas import tpu_sc as plsc

D_MODEL = 128
SCALE = math.sqrt(float(D_MODEL))
NUM_CORES = 2
NUM_SUBCORES = 16
NW = NUM_CORES * NUM_SUBCORES
CHUNK = 128
LANES = 16


@functools.partial(jax.jit, static_argnames=("n_chunks",))
def _lookup(idx, table, *, n_chunks):
    mesh = plsc.VectorSubcoreMesh(core_axis_name="c", subcore_axis_name="s")
    total_rows = NW * n_chunks * CHUNK

    @functools.partial(
        pl.kernel,
        mesh=mesh,
        out_type=jax.ShapeDtypeStruct((total_rows, D_MODEL), jnp.float32),
        scratch_types=[
            pltpu.VMEM((n_chunks, CHUNK), jnp.int32),
            pltpu.VMEM((CHUNK, D_MODEL), jnp.float32),
            pltpu.SemaphoreType.DMA,
        ],
    )
    def k(idx_hbm, table_hbm, out_hbm, idx_v, rows_v, sem):
        wid = lax.axis_index("s") * NUM_CORES + lax.axis_index("c")
        pltpu.sync_copy(idx_hbm.at[wid], idx_v)

        def chunk_body(j, carry):
            pltpu.async_copy(table_hbm.at[idx_v.at[j]], rows_v, sem).wait()

            def scale_row(r, c2):
                for c in range(D_MODEL // LANES):
                    sl = pl.ds(c * LANES, LANES)
                    rows_v[r, sl] = rows_v[r, sl] * SCALE
                return c2

            lax.fori_loop(0, CHUNK, scale_row, 0)
            base = (wid * n_chunks + j) * CHUNK
            pltpu.sync_copy(rows_v, out_hbm.at[pl.ds(base, CHUNK)])
            return carry

        lax.fori_loop(0, n_chunks, chunk_body, 0)

    return k(idx, table)


def kernel(x, table):
    b0, b1 = x.shape
    total = b0 * b1
    n_chunks = total // (NW * CHUNK)
    idx = x.astype(jnp.int32).reshape(NW, n_chunks, CHUNK)
    out = _lookup(idx, table, n_chunks=n_chunks)
    return out.reshape(b0, b1, D_MODEL)


# speedup vs baseline: 2.8830x; 1.1972x over previous
import functools
import math

import jax
import jax.numpy as jnp
from jax import lax
from jax.experimental import pallas as pl
from jax.experimental.pallas import tpu as pltpu
from jax.experimental.pallas import tpu_sc as plsc

D_MODEL = 128
SCALE = math.sqrt(float(D_MODEL))
NUM_CORES = 2
NUM_SUBCORES = 16
NW = NUM_CORES * NUM_SUBCORES
CHUNK = 128
LANES = 16
NBUF = 2


@functools.partial(jax.jit, static_argnames=("n_chunks",))
def _lookup(idx, table, *, n_chunks):
    assert n_chunks % NBUF == 0
    mesh = plsc.VectorSubcoreMesh(core_axis_name="c", subcore_axis_name="s")
    total_rows = NW * n_chunks * CHUNK

    @functools.partial(
        pl.kernel,
        mesh=mesh,
        out_type=jax.ShapeDtypeStruct((total_rows, D_MODEL), jnp.float32),
        scratch_types=[
            pltpu.VMEM((n_chunks, CHUNK), jnp.int32),
            pltpu.VMEM((CHUNK, D_MODEL), jnp.float32),
            pltpu.VMEM((CHUNK, D_MODEL), jnp.float32),
            pltpu.SemaphoreType.DMA,
            pltpu.SemaphoreType.DMA,
            pltpu.SemaphoreType.DMA,
            pltpu.SemaphoreType.DMA,
        ],
    )
    def k(idx_hbm, table_hbm, out_hbm, idx_v, rows0, rows1, g0, g1, s0, s1):
        wid = lax.axis_index("s") * NUM_CORES + lax.axis_index("c")
        pltpu.sync_copy(idx_hbm.at[wid], idx_v)
        rows = (rows0, rows1)
        gsem = (g0, g1)
        ssem = (s0, s1)
        out_base = wid * n_chunks

        pltpu.async_copy(table_hbm.at[idx_v.at[0]], rows0, g0)

        def steady(g, carry):
            for b in range(NBUF):
                j = NBUF * g + b
                nb = (b + 1) % NBUF

                @pl.when(j + 1 < n_chunks)
                def _():
                    @pl.when(j >= 1)
                    def _():
                        pltpu.make_async_copy(
                            rows[nb], out_hbm.at[pl.ds(0, CHUNK)], ssem[nb]
                        ).wait()

                    pltpu.async_copy(
                        table_hbm.at[idx_v.at[j + 1]], rows[nb], gsem[nb]
                    )

                pltpu.make_async_copy(
                    table_hbm.at[idx_v.at[j]], rows[b], gsem[b]
                ).wait()

                def scale_rows(r, c2, _b=b):
                    for rr in range(2):
                        for c in range(D_MODEL // LANES):
                            sl = pl.ds(c * LANES, LANES)
                            rows[_b][2 * r + rr, sl] = (
                                rows[_b][2 * r + rr, sl] * SCALE
                            )
                    return c2

                lax.fori_loop(0, CHUNK // 2, scale_rows, 0)

                pltpu.async_copy(
                    rows[b],
                    out_hbm.at[pl.ds((out_base + j) * CHUNK, CHUNK)],
                    ssem[b],
                )
            return carry

        lax.fori_loop(0, n_chunks // NBUF, steady, 0)

        for b in range(NBUF):
            pltpu.make_async_copy(
                rows[b], out_hbm.at[pl.ds(0, CHUNK)], ssem[b]
            ).wait()

    return k(idx, table)


def kernel(x, table):
    b0, b1 = x.shape
    total = b0 * b1
    n_chunks = total // (NW * CHUNK)
    idx = x.astype(jnp.int32).reshape(NW, n_chunks, CHUNK)
    out = _lookup(idx, table, n_chunks=n_chunks)
    return out.reshape(b0, b1, D_MODEL)


# speedup vs baseline: 4.8994x; 1.6994x over previous
import functools
import math

import jax
import jax.numpy as jnp
from jax import lax
from jax.experimental import pallas as pl
from jax.experimental.pallas import tpu as pltpu
from jax.experimental.pallas import tpu_sc as plsc

D_MODEL = 128
SCALE = math.sqrt(float(D_MODEL))
NUM_CORES = 2
NUM_SUBCORES = 16
NW = NUM_CORES * NUM_SUBCORES
ROWS_PER_CHUNK = 2
LANES = 16
NBUF = 2


@functools.partial(jax.jit, static_argnames=("b0", "b1"))
def _lookup(idx, table, *, b0, b1):
    mesh = plsc.VectorSubcoreMesh(core_axis_name="c", subcore_axis_name="s")
    chunk_idx = ROWS_PER_CHUNK * b1
    n_chunks = b0 // (NW * ROWS_PER_CHUNK)

    @functools.partial(
        pl.kernel,
        mesh=mesh,
        out_type=jax.ShapeDtypeStruct((b0, b1, D_MODEL), jnp.float32),
        scratch_types=[
            pltpu.VMEM((n_chunks, chunk_idx), jnp.int32),
            pltpu.VMEM((chunk_idx, D_MODEL), jnp.float32),
            pltpu.VMEM((chunk_idx, D_MODEL), jnp.float32),
            pltpu.VMEM((ROWS_PER_CHUNK, b1, D_MODEL), jnp.float32),
            pltpu.VMEM((ROWS_PER_CHUNK, b1, D_MODEL), jnp.float32),
            pltpu.SemaphoreType.DMA,
            pltpu.SemaphoreType.DMA,
            pltpu.SemaphoreType.DMA,
            pltpu.SemaphoreType.DMA,
        ],
        compiler_params=pltpu.CompilerParams(use_tc_tiling_on_sc=True),
    )
    def k(idx_hbm, table_hbm, out_hbm, idx_v, ga, gb, oa, ob, g0, g1, s0, s1):
        wid = lax.axis_index("s") * NUM_CORES + lax.axis_index("c")
        pltpu.sync_copy(idx_hbm.at[wid], idx_v)
        bufg = (ga, gb)
        bufo = (oa, ob)
        gsem = (g0, g1)
        ssem = (s0, s1)
        row_base = wid * n_chunks * ROWS_PER_CHUNK

        pltpu.async_copy(table_hbm.at[idx_v.at[0]], ga, g0)

        def steady(g, carry):
            for b in range(NBUF):
                j = NBUF * g + b
                nb = (b + 1) % NBUF

                @pl.when(j + 1 < n_chunks)
                def _():
                    @pl.when(j >= 1)
                    def _():
                        pltpu.make_async_copy(
                            bufo[nb],
                            out_hbm.at[pl.ds(0, ROWS_PER_CHUNK)],
                            ssem[nb],
                        ).wait()

                    pltpu.async_copy(
                        table_hbm.at[idx_v.at[j + 1]], bufg[nb], gsem[nb]
                    )

                pltpu.make_async_copy(
                    table_hbm.at[idx_v.at[j]], bufg[b], gsem[b]
                ).wait()

                def scale_rows(s, c2, _b=b):
                    for rr in range(ROWS_PER_CHUNK):
                        for c in range(D_MODEL // LANES):
                            sl = pl.ds(c * LANES, LANES)
                            bufo[_b][rr, s, sl] = bufg[_b][rr * b1 + s, sl] * SCALE
                    return c2

                lax.fori_loop(0, b1, scale_rows, 0)

                pltpu.async_copy(
                    bufo[b],
                    out_hbm.at[pl.ds(row_base + j * ROWS_PER_CHUNK, ROWS_PER_CHUNK)],
                    ssem[b],
                )
            return carry

        lax.fori_loop(0, n_chunks // NBUF, steady, 0)

        for b in range(NBUF):
            pltpu.make_async_copy(
                bufo[b], out_hbm.at[pl.ds(0, ROWS_PER_CHUNK)], ssem[b]
            ).wait()

    return k(idx, table)


def kernel(x, table):
    b0, b1 = x.shape
    n_chunks = b0 // (NW * ROWS_PER_CHUNK)
    idx = x.astype(jnp.int32).reshape(NW, n_chunks, ROWS_PER_CHUNK * b1)
    return _lookup(idx, table, b0=b0, b1=b1)


# speedup vs baseline: 8.5856x; 1.7524x over previous
import functools
import math

import jax
import jax.numpy as jnp
from jax import lax
from jax.experimental import pallas as pl
from jax.experimental.pallas import tpu as pltpu
from jax.experimental.pallas import tpu_sc as plsc

D_MODEL = 128
SCALE = math.sqrt(float(D_MODEL))
NUM_CORES = 2
NUM_SUBCORES = 16
NW = NUM_CORES * NUM_SUBCORES
CHUNK = 128
LANES = 16
NBUF = 2


@functools.partial(jax.jit, static_argnames=("n_chunks",))
def _lookup(idx, table, *, n_chunks):
    assert n_chunks % NBUF == 0
    mesh = plsc.VectorSubcoreMesh(core_axis_name="c", subcore_axis_name="s")
    total_rows = NW * n_chunks * CHUNK

    @functools.partial(
        pl.kernel,
        mesh=mesh,
        out_type=jax.ShapeDtypeStruct((total_rows, D_MODEL), jnp.float32),
        scratch_types=[
            pltpu.VMEM((n_chunks, CHUNK), jnp.int32),
            pltpu.VMEM((CHUNK, D_MODEL), jnp.float32),
            pltpu.VMEM((CHUNK, D_MODEL), jnp.float32),
            pltpu.SemaphoreType.DMA,
            pltpu.SemaphoreType.DMA,
            pltpu.SemaphoreType.DMA,
            pltpu.SemaphoreType.DMA,
        ],
    )
    def k(idx_hbm, table_hbm, out_hbm, idx_v, rows0, rows1, g0, g1, s0, s1):
        wid = lax.axis_index("s") * NUM_CORES + lax.axis_index("c")
        pltpu.sync_copy(idx_hbm.at[wid], idx_v)
        rows = (rows0, rows1)
        gsem = (g0, g1)
        ssem = (s0, s1)
        out_base = wid * n_chunks

        pltpu.async_copy(table_hbm.at[idx_v.at[0]], rows0, g0)

        def steady(g, carry):
            for b in range(NBUF):
                j = NBUF * g + b
                nb = (b + 1) % NBUF

                @pl.when(j + 1 < n_chunks)
                def _():
                    @pl.when(j >= 1)
                    def _():
                        pltpu.make_async_copy(
                            rows[nb], out_hbm.at[pl.ds(0, CHUNK)], ssem[nb]
                        ).wait()

                    pltpu.async_copy(
                        table_hbm.at[idx_v.at[j + 1]], rows[nb], gsem[nb]
                    )

                pltpu.make_async_copy(
                    table_hbm.at[idx_v.at[j]], rows[b], gsem[b]
                ).wait()

                def scale_rows(r, c2, _b=b):
                    for rr in range(2):
                        for c in range(D_MODEL // LANES):
                            sl = pl.ds(c * LANES, LANES)
                            rows[_b][2 * r + rr, sl] = (
                                rows[_b][2 * r + rr, sl] * SCALE
                            )
                    return c2

                lax.fori_loop(0, CHUNK // 2, scale_rows, 0)

                pltpu.async_copy(
                    rows[b],
                    out_hbm.at[pl.ds((out_base + j) * CHUNK, CHUNK)],
                    ssem[b],
                )
            return carry

        lax.fori_loop(0, n_chunks // NBUF, steady, 0)

        for b in range(NBUF):
            pltpu.make_async_copy(
                rows[b], out_hbm.at[pl.ds(0, CHUNK)], ssem[b]
            ).wait()

    return k(idx, table)


def kernel(x, table):
    b0, b1 = x.shape
    total = b0 * b1
    n_chunks = total // (NW * CHUNK)
    idx = x.astype(jnp.int32).T.reshape(NW, n_chunks, CHUNK)
    out = _lookup(idx, table, n_chunks=n_chunks)
    return out.reshape(b1, b0, D_MODEL).transpose(1, 0, 2)
